# baseline (speedup 1.0000x reference)
import jax
import jax.numpy as jnp
from jax.experimental import pallas as pl
from jax.experimental.pallas import tpu as pltpu

_HIDDEN = 2048
_NE = 16
_TOPK = 2
_INTER = 768
_TOKENS = 4096
_ASSIGN = _TOKENS * _TOPK
_TM = 256
_N_TILES = (_ASSIGN + _NE * (_TM - 1) + _TM - 1) // _TM
_M_PAD = _N_TILES * _TM
_ROUTER_BM = 512
_LANE = 128


def _router_body(x_ref, gw_ref, logits_ref, meta_ref):
    logits = jnp.dot(x_ref[...], gw_ref[...], preferred_element_type=jnp.float32)
    cols = jax.lax.broadcasted_iota(jnp.int32, logits.shape, 1)
    neg = jnp.float32(-jnp.inf)
    lm = jnp.where(cols < _NE, logits, neg)
    m0 = jnp.max(lm, axis=1, keepdims=True)
    i0 = jnp.min(jnp.where(lm == m0, cols, _NE), axis=1, keepdims=True)
    lm1 = jnp.where(cols == i0, neg, lm)
    m1 = jnp.max(lm1, axis=1, keepdims=True)
    i1 = jnp.min(jnp.where(lm1 == m1, cols, _NE), axis=1, keepdims=True)
    d = jnp.exp(m1 - m0)
    w0 = 1.0 / (1.0 + d)
    w1 = d / (1.0 + d)
    logits_ref[...] = logits
    meta = jnp.where(cols == 0, w0,
           jnp.where(cols == 1, w1,
           jnp.where(cols == 2, i0.astype(jnp.float32),
           jnp.where(cols == 3, i1.astype(jnp.float32), 0.0))))
    meta_ref[...] = meta


def _moe_body(te_ref, tv_ref, x_ref, g_ref, u_ref, d_ref, w_ref, out_ref):
    s = pl.program_id(0)

    @pl.when(tv_ref[s] == 1)
    def _():
        x = x_ref[...].astype(jnp.float32)
        g = jnp.dot(x, g_ref[0], preferred_element_type=jnp.float32)
        u = jnp.dot(x, u_ref[0], preferred_element_type=jnp.float32)
        sig = 1.0 / (1.0 + jnp.exp(-g))
        act = g * sig * u * w_ref[...]
        out = jnp.dot(act, d_ref[0], preferred_element_type=jnp.float32)
        out_ref[...] = out.astype(jnp.bfloat16)


def kernel(x, gate_w, gate_proj, up_proj, down_proj):
    x2 = x.reshape(-1, _HIDDEN)

    gw_pad = jnp.zeros((_HIDDEN, _LANE), jnp.float32).at[:, :_NE].set(gate_w)
    logits_pad, meta = pl.pallas_call(
        _router_body,
        grid=(_TOKENS // _ROUTER_BM,),
        in_specs=[
            pl.BlockSpec((_ROUTER_BM, _HIDDEN), lambda i: (i, 0)),
            pl.BlockSpec((_HIDDEN, _LANE), lambda i: (0, 0)),
        ],
        out_specs=[
            pl.BlockSpec((_ROUTER_BM, _LANE), lambda i: (i, 0)),
            pl.BlockSpec((_ROUTER_BM, _LANE), lambda i: (i, 0)),
        ],
        out_shape=[
            jax.ShapeDtypeStruct((_TOKENS, _LANE), jnp.float32),
            jax.ShapeDtypeStruct((_TOKENS, _LANE), jnp.float32),
        ],
    )(x2, gw_pad)

    router_logits = logits_pad[:, :_NE]
    rw = meta[:, :_TOPK]
    sel = meta[:, _TOPK:2 * _TOPK].astype(jnp.int32)

    sel_flat = sel.reshape(-1)
    onehot = (sel_flat[:, None] == jnp.arange(_NE)[None, :]).astype(jnp.int32)
    cum = jnp.cumsum(onehot, axis=0)
    counts = cum[-1]
    rank = jnp.take_along_axis(cum, sel_flat[:, None], axis=1)[:, 0] - 1
    padded = ((counts + _TM - 1) // _TM) * _TM
    bounds = jnp.cumsum(padded)
    pstart = bounds - padded
    pos = pstart[sel_flat] + rank

    tgrid = jnp.arange(_N_TILES, dtype=jnp.int32) * _TM
    tile_e = jnp.minimum(
        jnp.searchsorted(bounds, tgrid, side='right').astype(jnp.int32), _NE - 1)
    tile_v = (tgrid < bounds[-1]).astype(jnp.int32)

    tok_of = (jnp.arange(_ASSIGN) // _TOPK).astype(jnp.int32)
    inv = jnp.zeros((_M_PAD,), jnp.int32).at[pos].set(tok_of)
    x_sp = x2.astype(jnp.bfloat16)[inv]
    w_pos = jnp.zeros((_M_PAD, 1), jnp.float32).at[pos, 0].set(rw.reshape(-1))

    out_sp = pl.pallas_call(
        _moe_body,
        grid_spec=pltpu.PrefetchScalarGridSpec(
            num_scalar_prefetch=2,
            grid=(_N_TILES,),
            in_specs=[
                pl.BlockSpec((_TM, _HIDDEN), lambda s, te, tv: (s, 0)),
                pl.BlockSpec((1, _HIDDEN, _INTER), lambda s, te, tv: (te[s], 0, 0)),
                pl.BlockSpec((1, _HIDDEN, _INTER), lambda s, te, tv: (te[s], 0, 0)),
                pl.BlockSpec((1, _INTER, _HIDDEN), lambda s, te, tv: (te[s], 0, 0)),
                pl.BlockSpec((_TM, 1), lambda s, te, tv: (s, 0)),
            ],
            out_specs=pl.BlockSpec((_TM, _HIDDEN), lambda s, te, tv: (s, 0)),
        ),
        out_shape=jax.ShapeDtypeStruct((_M_PAD, _HIDDEN), jnp.bfloat16),
        compiler_params=pltpu.CompilerParams(
            dimension_semantics=("arbitrary",),
        ),
    )(tile_e, tile_v, x_sp, gate_proj, up_proj, down_proj, w_pos)

    pair = out_sp[pos].reshape(_TOKENS, _TOPK, _HIDDEN).astype(jnp.float32)
    final = pair.sum(axis=1)
    return (final.reshape(x.shape), router_logits)


# speedup vs baseline: 1.0099x; 1.0099x over previous
import jax
import jax.numpy as jnp
from jax.experimental import pallas as pl
from jax.experimental.pallas import tpu as pltpu

_HIDDEN = 2048
_NE = 16
_TOPK = 2
_INTER = 768
_TOKENS = 4096
_ASSIGN = _TOKENS * _TOPK
_TM = 256
_N_TILES = (_ASSIGN + _NE * (_TM - 1) + _TM - 1) // _TM
_M_PAD = _N_TILES * _TM
_ROUTER_BM = 512
_LANE = 128


def _router_body(x_ref, gw_ref, logits_ref, meta_ref):
    logits = jnp.dot(x_ref[...], gw_ref[...], preferred_element_type=jnp.float32)
    cols = jax.lax.broadcasted_iota(jnp.int32, logits.shape, 1)
    neg = jnp.float32(-jnp.inf)
    lm = jnp.where(cols < _NE, logits, neg)
    m0 = jnp.max(lm, axis=1, keepdims=True)
    i0 = jnp.min(jnp.where(lm == m0, cols, _NE), axis=1, keepdims=True)
    lm1 = jnp.where(cols == i0, neg, lm)
    m1 = jnp.max(lm1, axis=1, keepdims=True)
    i1 = jnp.min(jnp.where(lm1 == m1, cols, _NE), axis=1, keepdims=True)
    d = jnp.exp(m1 - m0)
    w0 = 1.0 / (1.0 + d)
    w1 = d / (1.0 + d)
    logits_ref[...] = logits
    meta = jnp.where(cols == 0, w0,
           jnp.where(cols == 1, w1,
           jnp.where(cols == 2, i0.astype(jnp.float32),
           jnp.where(cols == 3, i1.astype(jnp.float32), 0.0))))
    meta_ref[...] = meta


def _moe_body(te_ref, tv_ref, x_ref, g_ref, u_ref, d_ref, w_ref, out_ref):
    s = pl.program_id(0)

    @pl.when(tv_ref[s] == 1)
    def _():
        x = x_ref[...].astype(jnp.float32)
        g = jnp.dot(x, g_ref[0], preferred_element_type=jnp.float32)
        u = jnp.dot(x, u_ref[0], preferred_element_type=jnp.float32)
        sig = 1.0 / (1.0 + jnp.exp(-g))
        act = g * sig * u * w_ref[...]
        out = jnp.dot(act, d_ref[0], preferred_element_type=jnp.float32)
        out_ref[...] = out.astype(jnp.bfloat16)


def kernel(x, gate_w, gate_proj, up_proj, down_proj):
    x2 = x.reshape(-1, _HIDDEN)

    gw_pad = jnp.zeros((_HIDDEN, _LANE), jnp.float32).at[:, :_NE].set(gate_w)
    logits_pad, meta = pl.pallas_call(
        _router_body,
        grid=(_TOKENS // _ROUTER_BM,),
        in_specs=[
            pl.BlockSpec((_ROUTER_BM, _HIDDEN), lambda i: (i, 0)),
            pl.BlockSpec((_HIDDEN, _LANE), lambda i: (0, 0)),
        ],
        out_specs=[
            pl.BlockSpec((_ROUTER_BM, _LANE), lambda i: (i, 0)),
            pl.BlockSpec((_ROUTER_BM, _LANE), lambda i: (i, 0)),
        ],
        out_shape=[
            jax.ShapeDtypeStruct((_TOKENS, _LANE), jnp.float32),
            jax.ShapeDtypeStruct((_TOKENS, _LANE), jnp.float32),
        ],
    )(x2, gw_pad)

    router_logits = logits_pad[:, :_NE]
    rw = meta[:, :_TOPK]
    sel = meta[:, _TOPK:2 * _TOPK].astype(jnp.int32)

    sel_flat = sel.reshape(-1)
    onehot = (sel_flat[:, None] == jnp.arange(_NE)[None, :]).astype(jnp.int32)
    cum = jnp.cumsum(onehot, axis=0)
    counts = cum[-1]
    rank = jnp.take_along_axis(cum, sel_flat[:, None], axis=1)[:, 0] - 1
    padded = ((counts + _TM - 1) // _TM) * _TM
    bounds = jnp.cumsum(padded)
    pstart = bounds - padded
    pos = pstart[sel_flat] + rank

    tgrid = jnp.arange(_N_TILES, dtype=jnp.int32) * _TM
    tile_e = jnp.minimum(
        jnp.searchsorted(bounds, tgrid, side='right').astype(jnp.int32), _NE - 1)
    tile_v = (tgrid < bounds[-1]).astype(jnp.int32)

    tok_of = (jnp.arange(_ASSIGN) // _TOPK).astype(jnp.int32)
    inv = jnp.zeros((_M_PAD,), jnp.int32).at[pos].set(tok_of)
    x_sp = x2[inv]
    w_pos = jnp.zeros((_M_PAD, 1), jnp.float32).at[pos, 0].set(rw.reshape(-1))

    out_sp = pl.pallas_call(
        _moe_body,
        grid_spec=pltpu.PrefetchScalarGridSpec(
            num_scalar_prefetch=2,
            grid=(_N_TILES,),
            in_specs=[
                pl.BlockSpec((_TM, _HIDDEN), lambda s, te, tv: (s, 0)),
                pl.BlockSpec((1, _HIDDEN, _INTER), lambda s, te, tv: (te[s], 0, 0)),
                pl.BlockSpec((1, _HIDDEN, _INTER), lambda s, te, tv: (te[s], 0, 0)),
                pl.BlockSpec((1, _INTER, _HIDDEN), lambda s, te, tv: (te[s], 0, 0)),
                pl.BlockSpec((_TM, 1), lambda s, te, tv: (s, 0)),
            ],
            out_specs=pl.BlockSpec((_TM, _HIDDEN), lambda s, te, tv: (s, 0)),
        ),
        out_shape=jax.ShapeDtypeStruct((_M_PAD, _HIDDEN), jnp.bfloat16),
        compiler_params=pltpu.CompilerParams(
            dimension_semantics=("arbitrary",),
        ),
    )(tile_e, tile_v, x_sp, gate_proj, up_proj, down_proj, w_pos)

    pair = out_sp[pos].reshape(_TOKENS, _TOPK, _HIDDEN).astype(jnp.float32)
    final = pair.sum(axis=1)
    return (final.reshape(x.shape), router_logits)


# speedup vs baseline: 1.3882x; 1.3746x over previous
import functools

import jax
import jax.numpy as jnp
from jax import lax
from jax.experimental import pallas as pl
from jax.experimental.pallas import tpu as pltpu
from jax.experimental.pallas import tpu_sc as plsc

_HIDDEN = 2048
_NE = 16
_TOPK = 2
_INTER = 768
_TOKENS = 4096
_ASSIGN = _TOKENS * _TOPK
_TM = 256
_N_TILES = (_ASSIGN + _NE * (_TM - 1) + _TM - 1) // _TM
_M_PAD = _N_TILES * _TM
_ROUTER_BM = 512
_LANE = 128


def _router_body(x_ref, gw_ref, logits_ref, meta_ref):
    logits = jnp.dot(x_ref[...], gw_ref[...], preferred_element_type=jnp.float32)
    cols = jax.lax.broadcasted_iota(jnp.int32, logits.shape, 1)
    neg = jnp.float32(-jnp.inf)
    lm = jnp.where(cols < _NE, logits, neg)
    m0 = jnp.max(lm, axis=1, keepdims=True)
    i0 = jnp.min(jnp.where(lm == m0, cols, _NE), axis=1, keepdims=True)
    lm1 = jnp.where(cols == i0, neg, lm)
    m1 = jnp.max(lm1, axis=1, keepdims=True)
    i1 = jnp.min(jnp.where(lm1 == m1, cols, _NE), axis=1, keepdims=True)
    d = jnp.exp(m1 - m0)
    w0 = 1.0 / (1.0 + d)
    w1 = d / (1.0 + d)
    logits_ref[...] = logits
    meta = jnp.where(cols == 0, w0,
           jnp.where(cols == 1, w1,
           jnp.where(cols == 2, i0.astype(jnp.float32),
           jnp.where(cols == 3, i1.astype(jnp.float32), 0.0))))
    meta_ref[...] = meta


def _moe_body(te_ref, tv_ref, x_ref, g_ref, u_ref, d_ref, w_ref, out_ref):
    s = pl.program_id(0)

    @pl.when(tv_ref[s] == 1)
    def _():
        x = x_ref[...]
        g = jnp.dot(x, g_ref[0], preferred_element_type=jnp.float32)
        u = jnp.dot(x, u_ref[0], preferred_element_type=jnp.float32)
        sig = 1.0 / (1.0 + jnp.exp(-g))
        act = g * sig * u * w_ref[...]
        out_ref[...] = jnp.dot(act, d_ref[0], preferred_element_type=jnp.float32)


_NW = 32
_TPW = _TOKENS // _NW
_CT = 8
_VL = 16


def _combine_body(out_sp_hbm, pos_hbm, final_hbm, idx_v, rows_v, acc_v, sem):
    wid = lax.axis_index("s") * 2 + lax.axis_index("c")

    def chunk(c, carry):
        tbase = wid * _TPW + c * _CT
        pltpu.sync_copy(pos_hbm.at[pl.ds(2 * tbase, 2 * _CT)], idx_v)
        pltpu.async_copy(out_sp_hbm.at[idx_v], rows_v, sem).wait()

        def tok(t, carry2):
            def col(h, carry3):
                a = rows_v[2 * t, pl.ds(_VL * h, _VL)]
                b = rows_v[2 * t + 1, pl.ds(_VL * h, _VL)]
                acc_v[t, pl.ds(_VL * h, _VL)] = a + b
                return carry3
            return lax.fori_loop(0, _HIDDEN // _VL, col, carry2)
        lax.fori_loop(0, _CT, tok, carry)
        pltpu.sync_copy(acc_v, final_hbm.at[pl.ds(tbase, _CT)])
        return carry

    lax.fori_loop(0, _TPW // _CT, chunk, 0)


_sc_combine = functools.partial(
    pl.kernel,
    out_type=jax.ShapeDtypeStruct((_TOKENS, _HIDDEN), jnp.float32),
    mesh=plsc.VectorSubcoreMesh(core_axis_name="c", subcore_axis_name="s"),
    scratch_types=[
        pltpu.VMEM((2 * _CT,), jnp.int32),
        pltpu.VMEM((2 * _CT, _HIDDEN), jnp.float32),
        pltpu.VMEM((_CT, _HIDDEN), jnp.float32),
        pltpu.SemaphoreType.DMA,
    ],
)(_combine_body)


def kernel(x, gate_w, gate_proj, up_proj, down_proj):
    x2 = x.reshape(-1, _HIDDEN)

    gw_pad = jnp.zeros((_HIDDEN, _LANE), jnp.float32).at[:, :_NE].set(gate_w)
    logits_pad, meta = pl.pallas_call(
        _router_body,
        grid=(_TOKENS // _ROUTER_BM,),
        in_specs=[
            pl.BlockSpec((_ROUTER_BM, _HIDDEN), lambda i: (i, 0)),
            pl.BlockSpec((_HIDDEN, _LANE), lambda i: (0, 0)),
        ],
        out_specs=[
            pl.BlockSpec((_ROUTER_BM, _LANE), lambda i: (i, 0)),
            pl.BlockSpec((_ROUTER_BM, _LANE), lambda i: (i, 0)),
        ],
        out_shape=[
            jax.ShapeDtypeStruct((_TOKENS, _LANE), jnp.float32),
            jax.ShapeDtypeStruct((_TOKENS, _LANE), jnp.float32),
        ],
    )(x2, gw_pad)

    router_logits = logits_pad[:, :_NE]
    rw = meta[:, :_TOPK]
    sel = meta[:, _TOPK:2 * _TOPK].astype(jnp.int32)

    sel_flat = sel.reshape(-1)
    onehot = (sel_flat[:, None] == jnp.arange(_NE)[None, :]).astype(jnp.int32)
    cum = jnp.cumsum(onehot, axis=0)
    counts = cum[-1]
    rank = jnp.take_along_axis(cum, sel_flat[:, None], axis=1)[:, 0] - 1
    padded = ((counts + _TM - 1) // _TM) * _TM
    bounds = jnp.cumsum(padded)
    pstart = bounds - padded
    pos = pstart[sel_flat] + rank

    tgrid = jnp.arange(_N_TILES, dtype=jnp.int32) * _TM
    tile_e = jnp.minimum(
        jnp.searchsorted(bounds, tgrid, side='right').astype(jnp.int32), _NE - 1)
    tile_v = (tgrid < bounds[-1]).astype(jnp.int32)

    tok_of = (jnp.arange(_ASSIGN) // _TOPK).astype(jnp.int32)
    inv = jnp.zeros((_M_PAD,), jnp.int32).at[pos].set(tok_of)
    x_sp = x2[inv]
    w_pos = jnp.zeros((_M_PAD, 1), jnp.float32).at[pos, 0].set(rw.reshape(-1))

    out_sp = pl.pallas_call(
        _moe_body,
        grid_spec=pltpu.PrefetchScalarGridSpec(
            num_scalar_prefetch=2,
            grid=(_N_TILES,),
            in_specs=[
                pl.BlockSpec((_TM, _HIDDEN), lambda s, te, tv: (s, 0)),
                pl.BlockSpec((1, _HIDDEN, _INTER), lambda s, te, tv: (te[s], 0, 0)),
                pl.BlockSpec((1, _HIDDEN, _INTER), lambda s, te, tv: (te[s], 0, 0)),
                pl.BlockSpec((1, _INTER, _HIDDEN), lambda s, te, tv: (te[s], 0, 0)),
                pl.BlockSpec((_TM, 1), lambda s, te, tv: (s, 0)),
            ],
            out_specs=pl.BlockSpec((_TM, _HIDDEN), lambda s, te, tv: (s, 0)),
        ),
        out_shape=jax.ShapeDtypeStruct((_M_PAD, _HIDDEN), jnp.float32),
        compiler_params=pltpu.CompilerParams(
            dimension_semantics=("arbitrary",),
        ),
    )(tile_e, tile_v, x_sp, gate_proj, up_proj, down_proj, w_pos)

    final = _sc_combine(out_sp, pos)
    return (final.reshape(x.shape), router_logits)


# speedup vs baseline: 1.4739x; 1.0617x over previous
import functools

import jax
import jax.numpy as jnp
from jax import lax
from jax.experimental import pallas as pl
from jax.experimental.pallas import tpu as pltpu
from jax.experimental.pallas import tpu_sc as plsc

_HIDDEN = 2048
_NE = 16
_TOPK = 2
_INTER = 768
_TOKENS = 4096
_ASSIGN = _TOKENS * _TOPK
_TM = 256
_N_TILES = (_ASSIGN + _NE * (_TM - 1) + _TM - 1) // _TM
_M_PAD = _N_TILES * _TM
_ROUTER_BM = 512
_LANE = 128


def _router_body(x_ref, gw_ref, logits_ref, meta_ref):
    logits = jnp.dot(x_ref[...], gw_ref[...], preferred_element_type=jnp.float32)
    cols = jax.lax.broadcasted_iota(jnp.int32, logits.shape, 1)
    neg = jnp.float32(-jnp.inf)
    lm = jnp.where(cols < _NE, logits, neg)
    m0 = jnp.max(lm, axis=1, keepdims=True)
    i0 = jnp.min(jnp.where(lm == m0, cols, _NE), axis=1, keepdims=True)
    lm1 = jnp.where(cols == i0, neg, lm)
    m1 = jnp.max(lm1, axis=1, keepdims=True)
    i1 = jnp.min(jnp.where(lm1 == m1, cols, _NE), axis=1, keepdims=True)
    d = jnp.exp(m1 - m0)
    w0 = 1.0 / (1.0 + d)
    w1 = d / (1.0 + d)
    logits_ref[...] = logits
    meta = jnp.where(cols == 0, w0,
           jnp.where(cols == 1, w1,
           jnp.where(cols == 2, i0.astype(jnp.float32),
           jnp.where(cols == 3, i1.astype(jnp.float32), 0.0))))
    meta_ref[...] = meta


def _moe_body(te_ref, tv_ref, x_ref, g_ref, u_ref, d_ref, w_ref, out_ref):
    s = pl.program_id(0)

    @pl.when(tv_ref[s] == 1)
    def _():
        x = x_ref[...]
        g = jnp.dot(x, g_ref[0], preferred_element_type=jnp.float32)
        u = jnp.dot(x, u_ref[0], preferred_element_type=jnp.float32)
        sig = 1.0 / (1.0 + jnp.exp(-g))
        act = g * sig * u * w_ref[...]
        out_ref[...] = jnp.dot(act, d_ref[0], preferred_element_type=jnp.float32)


_NW = 32
_TPW = _TOKENS // _NW
_CT = 8
_VL = 16


def _combine_body(out_sp_hbm, pos_hbm, final_hbm, idx_v, rows0, rows1, acc_v, sem0, sem1):
    wid = lax.axis_index("s") * 2 + lax.axis_index("c")
    nch = _TPW // _CT

    pltpu.sync_copy(pos_hbm.at[pl.ds(wid * 2 * _TPW, 2 * _TPW)], idx_v)

    def fire(c, rows, sem):
        iv = idx_v[pl.ds(2 * _CT * c, 2 * _CT)]
        pltpu.async_copy(out_sp_hbm.at[iv], rows, sem)

    def drain_compute_store(c, rows, sem):
        pltpu.make_async_copy(out_sp_hbm.at[idx_v[pl.ds(0, 2 * _CT)]], rows, sem).wait()

        def tok(t, carry2):
            def col(h, carry3):
                a = rows[2 * t, pl.ds(_VL * h, _VL)]
                b = rows[2 * t + 1, pl.ds(_VL * h, _VL)]
                acc_v[t, pl.ds(_VL * h, _VL)] = a + b
                return carry3
            return lax.fori_loop(0, _HIDDEN // _VL, col, carry2)
        lax.fori_loop(0, _CT, tok, 0)
        pltpu.sync_copy(acc_v, final_hbm.at[pl.ds(wid * _TPW + c * _CT, _CT)])

    fire(0, rows0, sem0)

    def pairloop(cc, carry):
        c0 = 2 * cc
        fire(c0 + 1, rows1, sem1)
        drain_compute_store(c0, rows0, sem0)

        @pl.when(cc + 1 < nch // 2)
        def _():
            fire(c0 + 2, rows0, sem0)
        drain_compute_store(c0 + 1, rows1, sem1)
        return carry

    lax.fori_loop(0, nch // 2, pairloop, 0)


_sc_combine = functools.partial(
    pl.kernel,
    out_type=jax.ShapeDtypeStruct((_TOKENS, _HIDDEN), jnp.float32),
    mesh=plsc.VectorSubcoreMesh(core_axis_name="c", subcore_axis_name="s"),
    scratch_types=[
        pltpu.VMEM((2 * _TPW,), jnp.int32),
        pltpu.VMEM((2 * _CT, _HIDDEN), jnp.float32),
        pltpu.VMEM((2 * _CT, _HIDDEN), jnp.float32),
        pltpu.VMEM((_CT, _HIDDEN), jnp.float32),
        pltpu.SemaphoreType.DMA,
        pltpu.SemaphoreType.DMA,
    ],
)(_combine_body)


def kernel(x, gate_w, gate_proj, up_proj, down_proj):
    x2 = x.reshape(-1, _HIDDEN)

    gw_pad = jnp.zeros((_HIDDEN, _LANE), jnp.float32).at[:, :_NE].set(gate_w)
    logits_pad, meta = pl.pallas_call(
        _router_body,
        grid=(_TOKENS // _ROUTER_BM,),
        in_specs=[
            pl.BlockSpec((_ROUTER_BM, _HIDDEN), lambda i: (i, 0)),
            pl.BlockSpec((_HIDDEN, _LANE), lambda i: (0, 0)),
        ],
        out_specs=[
            pl.BlockSpec((_ROUTER_BM, _LANE), lambda i: (i, 0)),
            pl.BlockSpec((_ROUTER_BM, _LANE), lambda i: (i, 0)),
        ],
        out_shape=[
            jax.ShapeDtypeStruct((_TOKENS, _LANE), jnp.float32),
            jax.ShapeDtypeStruct((_TOKENS, _LANE), jnp.float32),
        ],
    )(x2, gw_pad)

    router_logits = logits_pad[:, :_NE]
    rw = meta[:, :_TOPK]
    sel = meta[:, _TOPK:2 * _TOPK].astype(jnp.int32)

    sel_flat = sel.reshape(-1)
    onehot = (sel_flat[:, None] == jnp.arange(_NE)[None, :]).astype(jnp.int32)
    cum = jnp.cumsum(onehot, axis=0)
    counts = cum[-1]
    rank = jnp.take_along_axis(cum, sel_flat[:, None], axis=1)[:, 0] - 1
    padded = ((counts + _TM - 1) // _TM) * _TM
    bounds = jnp.cumsum(padded)
    pstart = bounds - padded
    pos = pstart[sel_flat] + rank

    tgrid = jnp.arange(_N_TILES, dtype=jnp.int32) * _TM
    tile_e = jnp.minimum(
        jnp.searchsorted(bounds, tgrid, side='right').astype(jnp.int32), _NE - 1)
    tile_v = (tgrid < bounds[-1]).astype(jnp.int32)

    tok_of = (jnp.arange(_ASSIGN) // _TOPK).astype(jnp.int32)
    inv = jnp.zeros((_M_PAD,), jnp.int32).at[pos].set(tok_of)
    x_sp = x2[inv]
    w_pos = jnp.zeros((_M_PAD, 1), jnp.float32).at[pos, 0].set(rw.reshape(-1))

    out_sp = pl.pallas_call(
        _moe_body,
        grid_spec=pltpu.PrefetchScalarGridSpec(
            num_scalar_prefetch=2,
            grid=(_N_TILES,),
            in_specs=[
                pl.BlockSpec((_TM, _HIDDEN), lambda s, te, tv: (s, 0)),
                pl.BlockSpec((1, _HIDDEN, _INTER), lambda s, te, tv: (te[s], 0, 0)),
                pl.BlockSpec((1, _HIDDEN, _INTER), lambda s, te, tv: (te[s], 0, 0)),
                pl.BlockSpec((1, _INTER, _HIDDEN), lambda s, te, tv: (te[s], 0, 0)),
                pl.BlockSpec((_TM, 1), lambda s, te, tv: (s, 0)),
            ],
            out_specs=pl.BlockSpec((_TM, _HIDDEN), lambda s, te, tv: (s, 0)),
        ),
        out_shape=jax.ShapeDtypeStruct((_M_PAD, _HIDDEN), jnp.float32),
        compiler_params=pltpu.CompilerParams(
            dimension_semantics=("arbitrary",),
        ),
    )(tile_e, tile_v, x_sp, gate_proj, up_proj, down_proj, w_pos)

    final = _sc_combine(out_sp, pos)
    return (final.reshape(x.shape), router_logits)


# speedup vs baseline: 1.8819x; 1.2768x over previous
import functools

import jax
import jax.numpy as jnp
from jax import lax
from jax.experimental import pallas as pl
from jax.experimental.pallas import tpu as pltpu
from jax.experimental.pallas import tpu_sc as plsc

_HIDDEN = 2048
_NE = 16
_TOPK = 2
_INTER = 768
_TOKENS = 4096
_ASSIGN = _TOKENS * _TOPK
_TM = 256
_N_TILES = (_ASSIGN + _NE * (_TM - 1) + _TM - 1) // _TM
_M_PAD = _N_TILES * _TM
_ROUTER_BM = 512
_LANE = 128


def _router_body(x_ref, gw_ref, logits_ref, meta_ref):
    logits = jnp.dot(x_ref[...], gw_ref[...], preferred_element_type=jnp.float32)
    cols = jax.lax.broadcasted_iota(jnp.int32, logits.shape, 1)
    neg = jnp.float32(-jnp.inf)
    lm = jnp.where(cols < _NE, logits, neg)
    m0 = jnp.max(lm, axis=1, keepdims=True)
    i0 = jnp.min(jnp.where(lm == m0, cols, _NE), axis=1, keepdims=True)
    lm1 = jnp.where(cols == i0, neg, lm)
    m1 = jnp.max(lm1, axis=1, keepdims=True)
    i1 = jnp.min(jnp.where(lm1 == m1, cols, _NE), axis=1, keepdims=True)
    d = jnp.exp(m1 - m0)
    w0 = 1.0 / (1.0 + d)
    w1 = d / (1.0 + d)
    logits_ref[...] = logits
    meta = jnp.where(cols == 0, w0,
           jnp.where(cols == 1, w1,
           jnp.where(cols == 2, i0.astype(jnp.float32),
           jnp.where(cols == 3, i1.astype(jnp.float32), 0.0))))
    meta_ref[...] = meta


def _moe_body(te_ref, tv_ref, x_ref, g_ref, u_ref, d_ref, w_ref, out_ref):
    s = pl.program_id(0)

    @pl.when(tv_ref[s] == 1)
    def _():
        x = x_ref[...]
        g = jnp.dot(x, g_ref[0], preferred_element_type=jnp.float32)
        u = jnp.dot(x, u_ref[0], preferred_element_type=jnp.float32)
        sig = 1.0 / (1.0 + jnp.exp(-g))
        act = g * sig * u * w_ref[...]
        out_ref[...] = jnp.dot(act, d_ref[0], preferred_element_type=jnp.float32)


_NW = 32
_TPW = _TOKENS // _NW
_CT = 8
_VL = 16


def _combine_body(out_sp_hbm, pos_hbm, final_hbm, idx_v, rows0, rows1, acc_v, sem0, sem1):
    wid = lax.axis_index("s") * 2 + lax.axis_index("c")
    nch = _TPW // _CT

    pltpu.sync_copy(pos_hbm.at[pl.ds(wid * 2 * _TPW, 2 * _TPW)], idx_v)

    def fire(c, rows, sem):
        iv = idx_v[pl.ds(2 * _CT * c, 2 * _CT)]
        pltpu.async_copy(out_sp_hbm.at[iv], rows, sem)

    def drain_compute_store(c, rows, sem):
        pltpu.make_async_copy(out_sp_hbm.at[idx_v[pl.ds(0, 2 * _CT)]], rows, sem).wait()

        def tok(t, carry2):
            def col(h, carry3):
                a = rows[2 * t, pl.ds(_VL * h, _VL)]
                b = rows[2 * t + 1, pl.ds(_VL * h, _VL)]
                acc_v[t, pl.ds(_VL * h, _VL)] = a + b
                return carry3
            return lax.fori_loop(0, _HIDDEN // _VL, col, carry2)
        lax.fori_loop(0, _CT, tok, 0)
        pltpu.sync_copy(acc_v, final_hbm.at[pl.ds(wid * _TPW + c * _CT, _CT)])

    fire(0, rows0, sem0)

    def pairloop(cc, carry):
        c0 = 2 * cc
        fire(c0 + 1, rows1, sem1)
        drain_compute_store(c0, rows0, sem0)

        @pl.when(cc + 1 < nch // 2)
        def _():
            fire(c0 + 2, rows0, sem0)
        drain_compute_store(c0 + 1, rows1, sem1)
        return carry

    lax.fori_loop(0, nch // 2, pairloop, 0)


_RC = 16


def _xscatter_body(x2_hbm, pe_hbm, po_hbm, xsp_hbm, pe_v, po_v, rows0, rows1,
                   sem0, sem1, sems):
    wid = lax.axis_index("s") * 2 + lax.axis_index("c")
    nch = _TPW // _RC

    pltpu.sync_copy(pe_hbm.at[pl.ds(wid * _TPW, _TPW)], pe_v)
    pltpu.sync_copy(po_hbm.at[pl.ds(wid * _TPW, _TPW)], po_v)

    def fire_load(c, rows, sem):
        pltpu.async_copy(x2_hbm.at[pl.ds(wid * _TPW + _RC * c, _RC)], rows, sem)

    def scatter_out(c, rows, sem):
        pltpu.make_async_copy(x2_hbm.at[pl.ds(0, _RC)], rows, sem).wait()
        iv_e = pe_v[pl.ds(_RC * c, _RC)]
        iv_o = po_v[pl.ds(_RC * c, _RC)]
        pltpu.async_copy(rows, xsp_hbm.at[iv_e], sems)
        pltpu.async_copy(rows, xsp_hbm.at[iv_o], sems)
        pltpu.make_async_copy(rows, xsp_hbm.at[iv_e], sems).wait()
        pltpu.make_async_copy(rows, xsp_hbm.at[iv_o], sems).wait()

    fire_load(0, rows0, sem0)

    def pairloop(cc, carry):
        c0 = 2 * cc
        fire_load(c0 + 1, rows1, sem1)
        scatter_out(c0, rows0, sem0)

        @pl.when(cc + 1 < nch // 2)
        def _():
            fire_load(c0 + 2, rows0, sem0)
        scatter_out(c0 + 1, rows1, sem1)
        return carry

    lax.fori_loop(0, nch // 2, pairloop, 0)


_sc_xscatter = functools.partial(
    pl.kernel,
    out_type=jax.ShapeDtypeStruct((_M_PAD, _HIDDEN), jnp.float32),
    mesh=plsc.VectorSubcoreMesh(core_axis_name="c", subcore_axis_name="s"),
    scratch_types=[
        pltpu.VMEM((_TPW,), jnp.int32),
        pltpu.VMEM((_TPW,), jnp.int32),
        pltpu.VMEM((_RC, _HIDDEN), jnp.float32),
        pltpu.VMEM((_RC, _HIDDEN), jnp.float32),
        pltpu.SemaphoreType.DMA,
        pltpu.SemaphoreType.DMA,
        pltpu.SemaphoreType.DMA,
    ],
)(_xscatter_body)


_sc_combine = functools.partial(
    pl.kernel,
    out_type=jax.ShapeDtypeStruct((_TOKENS, _HIDDEN), jnp.float32),
    mesh=plsc.VectorSubcoreMesh(core_axis_name="c", subcore_axis_name="s"),
    scratch_types=[
        pltpu.VMEM((2 * _TPW,), jnp.int32),
        pltpu.VMEM((2 * _CT, _HIDDEN), jnp.float32),
        pltpu.VMEM((2 * _CT, _HIDDEN), jnp.float32),
        pltpu.VMEM((_CT, _HIDDEN), jnp.float32),
        pltpu.SemaphoreType.DMA,
        pltpu.SemaphoreType.DMA,
    ],
)(_combine_body)


def kernel(x, gate_w, gate_proj, up_proj, down_proj):
    x2 = x.reshape(-1, _HIDDEN)

    gw_pad = jnp.zeros((_HIDDEN, _LANE), jnp.float32).at[:, :_NE].set(gate_w)
    logits_pad, meta = pl.pallas_call(
        _router_body,
        grid=(_TOKENS // _ROUTER_BM,),
        in_specs=[
            pl.BlockSpec((_ROUTER_BM, _HIDDEN), lambda i: (i, 0)),
            pl.BlockSpec((_HIDDEN, _LANE), lambda i: (0, 0)),
        ],
        out_specs=[
            pl.BlockSpec((_ROUTER_BM, _LANE), lambda i: (i, 0)),
            pl.BlockSpec((_ROUTER_BM, _LANE), lambda i: (i, 0)),
        ],
        out_shape=[
            jax.ShapeDtypeStruct((_TOKENS, _LANE), jnp.float32),
            jax.ShapeDtypeStruct((_TOKENS, _LANE), jnp.float32),
        ],
    )(x2, gw_pad)

    router_logits = logits_pad[:, :_NE]
    rw = meta[:, :_TOPK]
    sel = meta[:, _TOPK:2 * _TOPK].astype(jnp.int32)

    sel_flat = sel.reshape(-1)
    onehot = (sel_flat[:, None] == jnp.arange(_NE)[None, :]).astype(jnp.int32)
    cum = jnp.cumsum(onehot, axis=0)
    counts = cum[-1]
    rank = jnp.take_along_axis(cum, sel_flat[:, None], axis=1)[:, 0] - 1
    padded = ((counts + _TM - 1) // _TM) * _TM
    bounds = jnp.cumsum(padded)
    pstart = bounds - padded
    pos = pstart[sel_flat] + rank

    tgrid = jnp.arange(_N_TILES, dtype=jnp.int32) * _TM
    tile_e = jnp.minimum(
        jnp.searchsorted(bounds, tgrid, side='right').astype(jnp.int32), _NE - 1)
    tile_v = (tgrid < bounds[-1]).astype(jnp.int32)

    pos_pair = pos.reshape(_TOKENS, _TOPK)
    x_sp = _sc_xscatter(x2, pos_pair[:, 0], pos_pair[:, 1])
    w_pos = jnp.zeros((_M_PAD, 1), jnp.float32).at[pos, 0].set(rw.reshape(-1))

    out_sp = pl.pallas_call(
        _moe_body,
        grid_spec=pltpu.PrefetchScalarGridSpec(
            num_scalar_prefetch=2,
            grid=(_N_TILES,),
            in_specs=[
                pl.BlockSpec((_TM, _HIDDEN), lambda s, te, tv: (s, 0)),
                pl.BlockSpec((1, _HIDDEN, _INTER), lambda s, te, tv: (te[s], 0, 0)),
                pl.BlockSpec((1, _HIDDEN, _INTER), lambda s, te, tv: (te[s], 0, 0)),
                pl.BlockSpec((1, _INTER, _HIDDEN), lambda s, te, tv: (te[s], 0, 0)),
                pl.BlockSpec((_TM, 1), lambda s, te, tv: (s, 0)),
            ],
            out_specs=pl.BlockSpec((_TM, _HIDDEN), lambda s, te, tv: (s, 0)),
        ),
        out_shape=jax.ShapeDtypeStruct((_M_PAD, _HIDDEN), jnp.float32),
        compiler_params=pltpu.CompilerParams(
            dimension_semantics=("arbitrary",),
        ),
    )(tile_e, tile_v, x_sp, gate_proj, up_proj, down_proj, w_pos)

    final = _sc_combine(out_sp, pos)
    return (final.reshape(x.shape), router_logits)


# speedup vs baseline: 2.1303x; 1.1320x over previous
import functools

import jax
import jax.numpy as jnp
from jax import lax
from jax.experimental import pallas as pl
from jax.experimental.pallas import tpu as pltpu
from jax.experimental.pallas import tpu_sc as plsc

_HIDDEN = 2048
_NE = 16
_TOPK = 2
_INTER = 768
_TOKENS = 4096
_ASSIGN = _TOKENS * _TOPK
_TM = 256
_N_TILES = (_ASSIGN + _NE * (_TM - 1) + _TM - 1) // _TM
_M_PAD = _N_TILES * _TM
_ROUTER_BM = 512
_LANE = 128


def _router_body(x_ref, gw_ref, logits_ref, meta_ref):
    logits = jnp.dot(x_ref[...], gw_ref[...], preferred_element_type=jnp.float32)
    cols = jax.lax.broadcasted_iota(jnp.int32, logits.shape, 1)
    neg = jnp.float32(-jnp.inf)
    lm = jnp.where(cols < _NE, logits, neg)
    m0 = jnp.max(lm, axis=1, keepdims=True)
    i0 = jnp.min(jnp.where(lm == m0, cols, _NE), axis=1, keepdims=True)
    lm1 = jnp.where(cols == i0, neg, lm)
    m1 = jnp.max(lm1, axis=1, keepdims=True)
    i1 = jnp.min(jnp.where(lm1 == m1, cols, _NE), axis=1, keepdims=True)
    d = jnp.exp(m1 - m0)
    w0 = 1.0 / (1.0 + d)
    w1 = d / (1.0 + d)
    logits_ref[...] = logits
    meta = jnp.where(cols == 0, w0,
           jnp.where(cols == 1, w1,
           jnp.where(cols == 2, i0.astype(jnp.float32),
           jnp.where(cols == 3, i1.astype(jnp.float32), 0.0))))
    meta_ref[...] = meta


def _moe_body(te_ref, tv_ref, x_ref, g_ref, u_ref, d_ref, w_ref, out_ref):
    s = pl.program_id(0)

    @pl.when(tv_ref[s] == 1)
    def _():
        x = x_ref[...]
        g = jnp.dot(x, g_ref[0], preferred_element_type=jnp.float32)
        u = jnp.dot(x, u_ref[0], preferred_element_type=jnp.float32)
        sig = 1.0 / (1.0 + jnp.exp(-g))
        act = g * sig * u * w_ref[...]
        out_ref[...] = jnp.dot(act, d_ref[0], preferred_element_type=jnp.float32)


_NW = 32
_TPW = _TOKENS // _NW
_CT = 8
_VL = 16


def _combine_body(out_sp_hbm, pos_hbm, final_hbm, idx_v, rows0, rows1, acc_v, sem0, sem1):
    wid = lax.axis_index("s") * 2 + lax.axis_index("c")
    nch = _TPW // _CT

    pltpu.sync_copy(pos_hbm.at[pl.ds(wid * 2 * _TPW, 2 * _TPW)], idx_v)

    def fire(c, rows, sem):
        iv = idx_v[pl.ds(2 * _CT * c, 2 * _CT)]
        pltpu.async_copy(out_sp_hbm.at[iv], rows, sem)

    def drain_compute_store(c, rows, sem):
        pltpu.make_async_copy(out_sp_hbm.at[idx_v[pl.ds(0, 2 * _CT)]], rows, sem).wait()

        nh = _HIDDEN // _VL

        @plsc.parallel_loop(0, _CT * nh, unroll=4)
        def _pairs(i):
            t = i // nh
            h = i - t * nh
            a = rows[2 * t, pl.ds(_VL * h, _VL)]
            b = rows[2 * t + 1, pl.ds(_VL * h, _VL)]
            acc_v[t, pl.ds(_VL * h, _VL)] = a + b
        pltpu.sync_copy(acc_v, final_hbm.at[pl.ds(wid * _TPW + c * _CT, _CT)])

    fire(0, rows0, sem0)

    def pairloop(cc, carry):
        c0 = 2 * cc
        fire(c0 + 1, rows1, sem1)
        drain_compute_store(c0, rows0, sem0)

        @pl.when(cc + 1 < nch // 2)
        def _():
            fire(c0 + 2, rows0, sem0)
        drain_compute_store(c0 + 1, rows1, sem1)
        return carry

    lax.fori_loop(0, nch // 2, pairloop, 0)


_RC = 16


def _xscatter_body(x2_hbm, pe_hbm, po_hbm, xsp_hbm, pe_v, po_v, rows0, rows1,
                   sem0, sem1, sems):
    wid = lax.axis_index("s") * 2 + lax.axis_index("c")
    nch = _TPW // _RC

    pltpu.sync_copy(pe_hbm.at[pl.ds(wid * _TPW, _TPW)], pe_v)
    pltpu.sync_copy(po_hbm.at[pl.ds(wid * _TPW, _TPW)], po_v)

    def fire_load(c, rows, sem):
        pltpu.async_copy(x2_hbm.at[pl.ds(wid * _TPW + _RC * c, _RC)], rows, sem)

    def scatter_out(c, rows, sem):
        pltpu.make_async_copy(x2_hbm.at[pl.ds(0, _RC)], rows, sem).wait()
        iv_e = pe_v[pl.ds(_RC * c, _RC)]
        iv_o = po_v[pl.ds(_RC * c, _RC)]
        pltpu.async_copy(rows, xsp_hbm.at[iv_e], sems)
        pltpu.async_copy(rows, xsp_hbm.at[iv_o], sems)
        pltpu.make_async_copy(rows, xsp_hbm.at[iv_e], sems).wait()
        pltpu.make_async_copy(rows, xsp_hbm.at[iv_o], sems).wait()

    fire_load(0, rows0, sem0)

    def pairloop(cc, carry):
        c0 = 2 * cc
        fire_load(c0 + 1, rows1, sem1)
        scatter_out(c0, rows0, sem0)

        @pl.when(cc + 1 < nch // 2)
        def _():
            fire_load(c0 + 2, rows0, sem0)
        scatter_out(c0 + 1, rows1, sem1)
        return carry

    lax.fori_loop(0, nch // 2, pairloop, 0)


_sc_xscatter = functools.partial(
    pl.kernel,
    out_type=jax.ShapeDtypeStruct((_M_PAD, _HIDDEN), jnp.float32),
    mesh=plsc.VectorSubcoreMesh(core_axis_name="c", subcore_axis_name="s"),
    scratch_types=[
        pltpu.VMEM((_TPW,), jnp.int32),
        pltpu.VMEM((_TPW,), jnp.int32),
        pltpu.VMEM((_RC, _HIDDEN), jnp.float32),
        pltpu.VMEM((_RC, _HIDDEN), jnp.float32),
        pltpu.SemaphoreType.DMA,
        pltpu.SemaphoreType.DMA,
        pltpu.SemaphoreType.DMA,
    ],
)(_xscatter_body)


_sc_combine = functools.partial(
    pl.kernel,
    out_type=jax.ShapeDtypeStruct((_TOKENS, _HIDDEN), jnp.float32),
    mesh=plsc.VectorSubcoreMesh(core_axis_name="c", subcore_axis_name="s"),
    scratch_types=[
        pltpu.VMEM((2 * _TPW,), jnp.int32),
        pltpu.VMEM((2 * _CT, _HIDDEN), jnp.float32),
        pltpu.VMEM((2 * _CT, _HIDDEN), jnp.float32),
        pltpu.VMEM((_CT, _HIDDEN), jnp.float32),
        pltpu.SemaphoreType.DMA,
        pltpu.SemaphoreType.DMA,
    ],
)(_combine_body)


def kernel(x, gate_w, gate_proj, up_proj, down_proj):
    x2 = x.reshape(-1, _HIDDEN)

    gw_pad = jnp.zeros((_HIDDEN, _LANE), jnp.float32).at[:, :_NE].set(gate_w)
    logits_pad, meta = pl.pallas_call(
        _router_body,
        grid=(_TOKENS // _ROUTER_BM,),
        in_specs=[
            pl.BlockSpec((_ROUTER_BM, _HIDDEN), lambda i: (i, 0)),
            pl.BlockSpec((_HIDDEN, _LANE), lambda i: (0, 0)),
        ],
        out_specs=[
            pl.BlockSpec((_ROUTER_BM, _LANE), lambda i: (i, 0)),
            pl.BlockSpec((_ROUTER_BM, _LANE), lambda i: (i, 0)),
        ],
        out_shape=[
            jax.ShapeDtypeStruct((_TOKENS, _LANE), jnp.float32),
            jax.ShapeDtypeStruct((_TOKENS, _LANE), jnp.float32),
        ],
    )(x2, gw_pad)

    router_logits = logits_pad[:, :_NE]
    rw = meta[:, :_TOPK]
    sel = meta[:, _TOPK:2 * _TOPK].astype(jnp.int32)

    sel_flat = sel.reshape(-1)
    onehot = (sel_flat[:, None] == jnp.arange(_NE)[None, :]).astype(jnp.int32)
    cum = jnp.cumsum(onehot, axis=0)
    counts = cum[-1]
    rank = jnp.take_along_axis(cum, sel_flat[:, None], axis=1)[:, 0] - 1
    padded = ((counts + _TM - 1) // _TM) * _TM
    bounds = jnp.cumsum(padded)
    pstart = bounds - padded
    pos = pstart[sel_flat] + rank

    tgrid = jnp.arange(_N_TILES, dtype=jnp.int32) * _TM
    tile_e = jnp.minimum(
        jnp.searchsorted(bounds, tgrid, side='right').astype(jnp.int32), _NE - 1)
    tile_v = (tgrid < bounds[-1]).astype(jnp.int32)

    pos_pair = pos.reshape(_TOKENS, _TOPK)
    x_sp = _sc_xscatter(x2, pos_pair[:, 0], pos_pair[:, 1])
    w_pos = jnp.zeros((_M_PAD, 1), jnp.float32).at[pos, 0].set(rw.reshape(-1))

    out_sp = pl.pallas_call(
        _moe_body,
        grid_spec=pltpu.PrefetchScalarGridSpec(
            num_scalar_prefetch=2,
            grid=(_N_TILES,),
            in_specs=[
                pl.BlockSpec((_TM, _HIDDEN), lambda s, te, tv: (s, 0)),
                pl.BlockSpec((1, _HIDDEN, _INTER), lambda s, te, tv: (te[s], 0, 0)),
                pl.BlockSpec((1, _HIDDEN, _INTER), lambda s, te, tv: (te[s], 0, 0)),
                pl.BlockSpec((1, _INTER, _HIDDEN), lambda s, te, tv: (te[s], 0, 0)),
                pl.BlockSpec((_TM, 1), lambda s, te, tv: (s, 0)),
            ],
            out_specs=pl.BlockSpec((_TM, _HIDDEN), lambda s, te, tv: (s, 0)),
        ),
        out_shape=jax.ShapeDtypeStruct((_M_PAD, _HIDDEN), jnp.float32),
        compiler_params=pltpu.CompilerParams(
            dimension_semantics=("arbitrary",),
        ),
    )(tile_e, tile_v, x_sp, gate_proj, up_proj, down_proj, w_pos)

    final = _sc_combine(out_sp, pos)
    return (final.reshape(x.shape), router_logits)
